# Initial kernel scaffold; baseline (speedup 1.0000x reference)
import functools

import jax
import jax.numpy as jnp
from jax import lax
from jax.experimental import pallas as pl
from jax.experimental.pallas import tpu as pltpu
from jax.experimental.pallas import tpu_sc as plsc

N_NODES = 100000
N_EDGES = 6400000

NPAD = 102400
EPAD = 6553600
EROWS = EPAD // 128
NWORK = 32
ROWS_W = EROWS // NWORK
WIN = 16
NWIN = ROWS_W // WIN
SLICE = NPAD // 16


def _mesh():
    return plsc.VectorSubcoreMesh(core_axis_name="c", subcore_axis_name="s")


def _deg_body(dst_hbm, ones_hbm, zeros_hbm, out_hbm, dwin, ones_v, acc):
    cid = lax.axis_index("c")
    sid = lax.axis_index("s")
    wid = cid * 16 + sid
    pltpu.sync_copy(ones_hbm, ones_v)
    pltpu.sync_copy(zeros_hbm.at[pl.ds(sid * SLICE, SLICE)],
                    acc.at[pl.ds(sid * SLICE, SLICE)])
    plsc.subcore_barrier()

    def window(w, carry):
        row0 = wid * ROWS_W + w * WIN
        pltpu.sync_copy(dst_hbm.at[pl.ds(row0, WIN)], dwin)
        for j in range(WIN):
            pltpu.sync_copy(ones_v, acc.at[dwin.at[j]], add=True)
        return carry

    lax.fori_loop(0, NWIN, window, 0)
    plsc.subcore_barrier()
    pltpu.sync_copy(acc.at[pl.ds(sid * SLICE, SLICE)],
                    out_hbm.at[pl.ds(cid * NPAD + sid * SLICE, SLICE)])


@jax.jit
def _deg_call(dst_p, ones128, zeros1):
    k = functools.partial(
        pl.kernel,
        out_type=jax.ShapeDtypeStruct((2 * NPAD,), jnp.float32),
        mesh=_mesh(),
        scratch_types=[
            pltpu.VMEM((WIN, 128), jnp.int32),
            pltpu.VMEM((128,), jnp.float32),
            pltpu.VMEM_SHARED((NPAD,), jnp.float32),
        ],
    )(_deg_body)
    return k(dst_p, ones128, zeros1).reshape(2, NPAD)


def _agg_body(src_hbm, dst_hbm, table_hbm, zeros_hbm, out_hbm,
              swin, dwin, msg, tab, acc, sem):
    cid = lax.axis_index("c")
    sid = lax.axis_index("s")
    wid = cid * 16 + sid
    pltpu.sync_copy(table_hbm.at[pl.ds(sid * SLICE, SLICE)],
                    tab.at[pl.ds(sid * SLICE, SLICE)])
    pltpu.sync_copy(zeros_hbm.at[pl.ds(sid * SLICE, SLICE)],
                    acc.at[pl.ds(sid * SLICE, SLICE)])
    plsc.subcore_barrier()

    def window(w, carry):
        row0 = wid * ROWS_W + w * WIN
        pltpu.sync_copy(src_hbm.at[pl.ds(row0, WIN)], swin)
        pltpu.sync_copy(dst_hbm.at[pl.ds(row0, WIN)], dwin)
        cps = [pltpu.async_copy(tab.at[swin.at[j]], msg.at[j], sem)
               for j in range(WIN)]
        for c in cps:
            c.wait()
        for j in range(WIN):
            pltpu.sync_copy(msg.at[j], acc.at[dwin.at[j]], add=True)
        return carry

    lax.fori_loop(0, NWIN, window, 0)
    plsc.subcore_barrier()
    pltpu.sync_copy(acc.at[pl.ds(sid * SLICE, SLICE)],
                    out_hbm.at[pl.ds(cid * NPAD + sid * SLICE, SLICE)])


@jax.jit
def _agg_call(src_p, dst_p, table, zeros2):
    k = functools.partial(
        pl.kernel,
        out_type=jax.ShapeDtypeStruct((2 * NPAD, 2), jnp.float32),
        mesh=_mesh(),
        scratch_types=[
            pltpu.VMEM((WIN, 128), jnp.int32),
            pltpu.VMEM((WIN, 128), jnp.int32),
            pltpu.VMEM((WIN, 128, 2), jnp.float32),
            pltpu.VMEM_SHARED((NPAD, 2), jnp.float32),
            pltpu.VMEM_SHARED((NPAD, 2), jnp.float32),
            pltpu.SemaphoreType.DMA,
        ],
    )(_agg_body)
    return k(src_p, dst_p, table, zeros2).reshape(2, NPAD, 2)


def _dense1_body(degp_ref, xt_ref, dis_ref, s1_ref):
    deg = degp_ref[0:1, :] + degp_ref[1:2, :] + 1.0
    dis = lax.rsqrt(deg)
    dis_ref[...] = dis
    s1_ref[...] = xt_ref[...] * dis


@jax.jit
def _dense1(degp, xt):
    return pl.pallas_call(
        _dense1_body,
        out_shape=(jax.ShapeDtypeStruct((1, NPAD), jnp.float32),
                   jax.ShapeDtypeStruct((2, NPAD), jnp.float32)),
    )(degp, xt)


def _dense2_body(ap_ref, s1_ref, dis_ref, w1_ref, b1_ref, w2_ref, out_ref):
    dis = dis_ref[...]
    a = (ap_ref[0] + ap_ref[1] + s1_ref[...]) * dis
    a0 = a[0:1, :]
    a1 = a[1:2, :]
    t0 = jnp.zeros_like(a0)
    t1 = jnp.zeros_like(a0)
    for j in range(16):
        hj = jnp.maximum(a0 * w1_ref[0, j] + a1 * w1_ref[1, j] + b1_ref[j],
                         0.0)
        t0 = t0 + hj * w2_ref[j, 0]
        t1 = t1 + hj * w2_ref[j, 1]
    out_ref[0:1, :] = t0 * dis
    out_ref[1:2, :] = t1 * dis


@jax.jit
def _dense2(ap, s1t, dis, W1, b1, W2):
    return pl.pallas_call(
        _dense2_body,
        in_specs=[
            pl.BlockSpec(memory_space=pltpu.ANY),
            pl.BlockSpec(memory_space=pltpu.ANY),
            pl.BlockSpec(memory_space=pltpu.ANY),
            pl.BlockSpec(memory_space=pltpu.SMEM),
            pl.BlockSpec(memory_space=pltpu.SMEM),
            pl.BlockSpec(memory_space=pltpu.SMEM),
        ],
        out_shape=jax.ShapeDtypeStruct((2, NPAD), jnp.float32),
    )(ap, s1t, dis, W1, b1, W2)


def _dense3_body(ap_ref, s2_ref, dis_ref, b2_ref, out_ref):
    z = (ap_ref[0] + ap_ref[1] + s2_ref[...]) * dis_ref[...]
    z0 = z[0:1, :] + b2_ref[0]
    z1 = z[1:2, :] + b2_ref[1]
    m = jnp.maximum(z0, z1)
    e0 = jnp.exp(z0 - m)
    e1 = jnp.exp(z1 - m)
    s = e0 + e1
    out_ref[0:1, :] = e0 / s
    out_ref[1:2, :] = e1 / s


@jax.jit
def _dense3(ap, s2t, dis, b2):
    return pl.pallas_call(
        _dense3_body,
        in_specs=[
            pl.BlockSpec(memory_space=pltpu.ANY),
            pl.BlockSpec(memory_space=pltpu.ANY),
            pl.BlockSpec(memory_space=pltpu.ANY),
            pl.BlockSpec(memory_space=pltpu.SMEM),
        ],
        out_shape=jax.ShapeDtypeStruct((2, NPAD), jnp.float32),
    )(ap, s2t, dis, b2)


def kernel(x, edge_index, W1, b1, W2, b2):
    src = edge_index[0].astype(jnp.int32)
    dst = edge_index[1].astype(jnp.int32)
    pad_ids = (N_NODES
               + jnp.arange(EPAD - N_EDGES, dtype=jnp.int32)
               % (NPAD - N_NODES))
    src_p = jnp.concatenate([src, pad_ids]).reshape(EROWS, 128)
    dst_p = jnp.concatenate([dst, pad_ids]).reshape(EROWS, 128)
    xt = jnp.pad(x, ((0, NPAD - N_NODES), (0, 0))).T
    zeros1 = jnp.zeros((NPAD,), jnp.float32)
    zeros2 = jnp.zeros((NPAD, 2), jnp.float32)
    ones128 = jnp.ones((128,), jnp.float32)

    degp = _deg_call(dst_p, ones128, zeros1)
    dis, s1t = _dense1(degp, xt)
    agg1 = _agg_call(src_p, dst_p, s1t.T, zeros2)
    s2t = _dense2(agg1.transpose(0, 2, 1), s1t, dis, W1, b1, W2)
    agg2 = _agg_call(src_p, dst_p, s2t.T, zeros2)
    outt = _dense3(agg2.transpose(0, 2, 1), s2t, dis, b2)
    return outt.T[:N_NODES]


# speedup vs baseline: 101.5534x; 101.5534x over previous
import functools

import jax
import jax.numpy as jnp
from jax import lax
from jax.experimental import pallas as pl
from jax.experimental.pallas import tpu as pltpu
from jax.experimental.pallas import tpu_sc as plsc

N_NODES = 100000
N_EDGES = 6400000

NPAD = 102400
EPAD = 6553600
EROWS = EPAD // 128
NWORK = 32
ROWS_W = EROWS // NWORK
WIN = 16
NWIN = ROWS_W // WIN
SLICE = NPAD // 16


def _mesh():
    return plsc.VectorSubcoreMesh(core_axis_name="c", subcore_axis_name="s")


def _deg_body(dst_hbm, ones_hbm, zeros_hbm, out_hbm, dwin, ones_v, acc):
    cid = lax.axis_index("c")
    sid = lax.axis_index("s")
    wid = cid * 16 + sid
    pltpu.sync_copy(ones_hbm, ones_v)
    pltpu.sync_copy(zeros_hbm.at[pl.ds(sid * SLICE, SLICE)],
                    acc.at[pl.ds(sid * SLICE, SLICE)])
    plsc.subcore_barrier()

    def window(w, carry):
        row0 = wid * ROWS_W + w * WIN
        pltpu.sync_copy(dst_hbm.at[pl.ds(row0, WIN)], dwin)
        for j in range(WIN):
            pltpu.sync_copy(ones_v, acc.at[dwin.at[j]], add=True)
        return carry

    lax.fori_loop(0, NWIN, window, 0)
    plsc.subcore_barrier()
    pltpu.sync_copy(acc.at[pl.ds(sid * SLICE, SLICE)],
                    out_hbm.at[pl.ds(cid * NPAD + sid * SLICE, SLICE)])


@jax.jit
def _deg_call(dst_p, ones128, zeros1):
    k = functools.partial(
        pl.kernel,
        out_type=jax.ShapeDtypeStruct((2 * NPAD,), jnp.float32),
        mesh=_mesh(),
        scratch_types=[
            pltpu.VMEM((WIN, 128), jnp.int32),
            pltpu.VMEM((128,), jnp.float32),
            pltpu.VMEM_SHARED((NPAD,), jnp.float32),
        ],
    )(_deg_body)
    return k(dst_p, ones128, zeros1).reshape(2, NPAD)


def _agg_body(src_hbm, dst_hbm, tab0_hbm, tab1_hbm, zeros_hbm, out_hbm,
              swin, dwin, msg0, msg1, tab0, tab1, acc0, acc1, sem):
    cid = lax.axis_index("c")
    sid = lax.axis_index("s")
    wid = cid * 16 + sid
    sl = pl.ds(sid * SLICE, SLICE)
    pltpu.sync_copy(tab0_hbm.at[sl], tab0.at[sl])
    pltpu.sync_copy(tab1_hbm.at[sl], tab1.at[sl])
    pltpu.sync_copy(zeros_hbm.at[sl], acc0.at[sl])
    pltpu.sync_copy(zeros_hbm.at[sl], acc1.at[sl])
    plsc.subcore_barrier()

    def window(w, carry):
        row0 = wid * ROWS_W + w * WIN
        pltpu.sync_copy(src_hbm.at[pl.ds(row0, WIN)], swin)
        pltpu.sync_copy(dst_hbm.at[pl.ds(row0, WIN)], dwin)
        cps = []
        for j in range(WIN):
            cps.append(pltpu.async_copy(tab0.at[swin.at[j]], msg0.at[j], sem))
            cps.append(pltpu.async_copy(tab1.at[swin.at[j]], msg1.at[j], sem))
        for c in cps:
            c.wait()
        for j in range(WIN):
            pltpu.sync_copy(msg0.at[j], acc0.at[dwin.at[j]], add=True)
            pltpu.sync_copy(msg1.at[j], acc1.at[dwin.at[j]], add=True)
        return carry

    lax.fori_loop(0, NWIN, window, 0)
    plsc.subcore_barrier()
    base = cid * 2 * NPAD + sid * SLICE
    pltpu.sync_copy(acc0.at[sl], out_hbm.at[pl.ds(base, SLICE)])
    pltpu.sync_copy(acc1.at[sl], out_hbm.at[pl.ds(base + NPAD, SLICE)])


@jax.jit
def _agg_call(src_p, dst_p, tabc0, tabc1, zeros1):
    k = functools.partial(
        pl.kernel,
        out_type=jax.ShapeDtypeStruct((4 * NPAD,), jnp.float32),
        mesh=_mesh(),
        scratch_types=[
            pltpu.VMEM((WIN, 128), jnp.int32),
            pltpu.VMEM((WIN, 128), jnp.int32),
            pltpu.VMEM((WIN, 128), jnp.float32),
            pltpu.VMEM((WIN, 128), jnp.float32),
            pltpu.VMEM_SHARED((NPAD,), jnp.float32),
            pltpu.VMEM_SHARED((NPAD,), jnp.float32),
            pltpu.VMEM_SHARED((NPAD,), jnp.float32),
            pltpu.VMEM_SHARED((NPAD,), jnp.float32),
            pltpu.SemaphoreType.DMA,
        ],
    )(_agg_body)
    return k(src_p, dst_p, tabc0, tabc1, zeros1).reshape(2, 2, NPAD)


def _dense1_body(degp_ref, xt_ref, dis_ref, s1_ref):
    deg = degp_ref[0:1, :] + degp_ref[1:2, :] + 1.0
    dis = lax.rsqrt(deg)
    dis_ref[...] = dis
    s1_ref[...] = xt_ref[...] * dis


@jax.jit
def _dense1(degp, xt):
    return pl.pallas_call(
        _dense1_body,
        out_shape=(jax.ShapeDtypeStruct((1, NPAD), jnp.float32),
                   jax.ShapeDtypeStruct((2, NPAD), jnp.float32)),
    )(degp, xt)


def _dense2_body(ap_ref, s1_ref, dis_ref, w1_ref, b1_ref, w2_ref, out_ref):
    dis = dis_ref[...]
    a = (ap_ref[0] + ap_ref[1] + s1_ref[...]) * dis
    a0 = a[0:1, :]
    a1 = a[1:2, :]
    t0 = jnp.zeros_like(a0)
    t1 = jnp.zeros_like(a0)
    for j in range(16):
        hj = jnp.maximum(a0 * w1_ref[0, j] + a1 * w1_ref[1, j] + b1_ref[j],
                         0.0)
        t0 = t0 + hj * w2_ref[j, 0]
        t1 = t1 + hj * w2_ref[j, 1]
    out_ref[0:1, :] = t0 * dis
    out_ref[1:2, :] = t1 * dis


@jax.jit
def _dense2(ap, s1t, dis, W1, b1, W2):
    return pl.pallas_call(
        _dense2_body,
        in_specs=[
            pl.BlockSpec(memory_space=pltpu.VMEM),
            pl.BlockSpec(memory_space=pltpu.VMEM),
            pl.BlockSpec(memory_space=pltpu.VMEM),
            pl.BlockSpec(memory_space=pltpu.SMEM),
            pl.BlockSpec(memory_space=pltpu.SMEM),
            pl.BlockSpec(memory_space=pltpu.SMEM),
        ],
        out_shape=jax.ShapeDtypeStruct((2, NPAD), jnp.float32),
    )(ap, s1t, dis, W1, b1, W2)


def _dense3_body(ap_ref, s2_ref, dis_ref, b2_ref, out_ref):
    z = (ap_ref[0] + ap_ref[1] + s2_ref[...]) * dis_ref[...]
    z0 = z[0:1, :] + b2_ref[0]
    z1 = z[1:2, :] + b2_ref[1]
    m = jnp.maximum(z0, z1)
    e0 = jnp.exp(z0 - m)
    e1 = jnp.exp(z1 - m)
    s = e0 + e1
    out_ref[0:1, :] = e0 / s
    out_ref[1:2, :] = e1 / s


@jax.jit
def _dense3(ap, s2t, dis, b2):
    return pl.pallas_call(
        _dense3_body,
        in_specs=[
            pl.BlockSpec(memory_space=pltpu.VMEM),
            pl.BlockSpec(memory_space=pltpu.VMEM),
            pl.BlockSpec(memory_space=pltpu.VMEM),
            pl.BlockSpec(memory_space=pltpu.SMEM),
        ],
        out_shape=jax.ShapeDtypeStruct((2, NPAD), jnp.float32),
    )(ap, s2t, dis, b2)


def kernel(x, edge_index, W1, b1, W2, b2):
    src = edge_index[0].astype(jnp.int32)
    dst = edge_index[1].astype(jnp.int32)
    pad_ids = (N_NODES
               + jnp.arange(EPAD - N_EDGES, dtype=jnp.int32)
               % (NPAD - N_NODES))
    src_p = jnp.concatenate([src, pad_ids]).reshape(EROWS, 128)
    dst_p = jnp.concatenate([dst, pad_ids]).reshape(EROWS, 128)
    xt = jnp.pad(x, ((0, NPAD - N_NODES), (0, 0))).T
    zeros1 = jnp.zeros((NPAD,), jnp.float32)
    ones128 = jnp.ones((128,), jnp.float32)

    degp = _deg_call(dst_p, ones128, zeros1)
    dis, s1t = _dense1(degp, xt)
    agg1 = _agg_call(src_p, dst_p, s1t[0], s1t[1], zeros1)
    s2t = _dense2(agg1, s1t, dis, W1, b1, W2)
    agg2 = _agg_call(src_p, dst_p, s2t[0], s2t[1], zeros1)
    outt = _dense3(agg2, s2t, dis, b2)
    return outt.T[:N_NODES]


# speedup vs baseline: 179.9329x; 1.7718x over previous
import functools

import jax
import jax.numpy as jnp
from jax import lax
from jax.experimental import pallas as pl
from jax.experimental.pallas import tpu as pltpu
from jax.experimental.pallas import tpu_sc as plsc

N_NODES = 100000
N_EDGES = 6400000

NPAD = 102400
EPAD = 6553600
EROWS = EPAD // 128
NWORK = 32
ROWS_W = EROWS // NWORK
WIN = 16
NWIN = ROWS_W // WIN
SLICE = NPAD // 16


def _mesh():
    return plsc.VectorSubcoreMesh(core_axis_name="c", subcore_axis_name="s")


def _deg_body(dst_hbm, ones_hbm, zeros_hbm, out_hbm, dwin, ones_v, acc,
              semi, sems):
    cid = lax.axis_index("c")
    sid = lax.axis_index("s")
    wid = cid * 16 + sid
    pltpu.sync_copy(ones_hbm, ones_v)
    pltpu.sync_copy(zeros_hbm.at[pl.ds(sid * SLICE, SLICE)],
                    acc.at[pl.ds(sid * SLICE, SLICE)])
    plsc.subcore_barrier()

    def idx_load(w, slot):
        row0 = wid * ROWS_W + w * WIN
        pltpu.async_copy(dst_hbm.at[pl.ds(row0, WIN)], dwin.at[slot],
                         semi.at[slot])

    def idx_wait(slot):
        pltpu.make_async_copy(dst_hbm.at[pl.ds(0, WIN)], dwin.at[slot],
                              semi.at[slot]).wait()

    def drain_scat(par):
        for j in range(WIN):
            pltpu.make_async_copy(zeros_hbm.at[pl.ds(0, 128)],
                                  ones_v, sems.at[par]).wait()

    def scatters(slot, par):
        for j in range(WIN):
            pltpu.async_copy(ones_v, acc.at[dwin.at[slot, j]], sems.at[par],
                             add=True)

    def win(w, slot, par, drain, load):
        if drain:
            drain_scat(par)
        if load:
            idx_load(w + 2, (slot + 2) % 4)
        idx_wait(slot)
        scatters(slot, par)

    idx_load(0, 0)
    idx_load(1, 1)
    win(0, 0, 0, drain=False, load=True)
    win(1, 1, 1, drain=False, load=True)

    def body(g, carry):
        w0 = 2 + 4 * g
        for b in range(4):
            win(w0 + b, (2 + b) % 4, b % 2, drain=True, load=True)
        return carry

    lax.fori_loop(0, (NWIN - 4) // 4, body, 0)
    win(NWIN - 2, (NWIN - 2) % 4, 0, drain=True, load=False)
    win(NWIN - 1, (NWIN - 1) % 4, 1, drain=True, load=False)
    drain_scat(0)
    drain_scat(1)

    plsc.subcore_barrier()
    pltpu.sync_copy(acc.at[pl.ds(sid * SLICE, SLICE)],
                    out_hbm.at[pl.ds(cid * NPAD + sid * SLICE, SLICE)])


@jax.jit
def _deg_call(dst_p, ones128, zeros1):
    k = functools.partial(
        pl.kernel,
        out_type=jax.ShapeDtypeStruct((2 * NPAD,), jnp.float32),
        mesh=_mesh(),
        scratch_types=[
            pltpu.VMEM((4, WIN, 128), jnp.int32),
            pltpu.VMEM((128,), jnp.float32),
            pltpu.VMEM_SHARED((NPAD,), jnp.float32),
            pltpu.SemaphoreType.DMA((4,)),
            pltpu.SemaphoreType.DMA((2,)),
        ],
    )(_deg_body)
    return k(dst_p, ones128, zeros1).reshape(2, NPAD)


def _agg_body(src_hbm, dst_hbm, tab0_hbm, tab1_hbm, zeros_hbm, out_hbm,
              swin, dwin, msg0, msg1, tab0, tab1, acc0, acc1,
              semi, semg, sems):
    cid = lax.axis_index("c")
    sid = lax.axis_index("s")
    wid = cid * 16 + sid
    sl = pl.ds(sid * SLICE, SLICE)
    pltpu.sync_copy(tab0_hbm.at[sl], tab0.at[sl])
    pltpu.sync_copy(tab1_hbm.at[sl], tab1.at[sl])
    pltpu.sync_copy(zeros_hbm.at[sl], acc0.at[sl])
    pltpu.sync_copy(zeros_hbm.at[sl], acc1.at[sl])
    plsc.subcore_barrier()

    def idx_load(w, slot):
        row0 = wid * ROWS_W + w * WIN
        pltpu.async_copy(src_hbm.at[pl.ds(row0, WIN)], swin.at[slot],
                         semi.at[slot])
        pltpu.async_copy(dst_hbm.at[pl.ds(row0, WIN)], dwin.at[slot],
                         semi.at[slot])

    def idx_wait(slot):
        pltpu.make_async_copy(src_hbm.at[pl.ds(0, WIN)], swin.at[slot],
                              semi.at[slot]).wait()
        pltpu.make_async_copy(dst_hbm.at[pl.ds(0, WIN)], dwin.at[slot],
                              semi.at[slot]).wait()

    def drain_scat(par):
        for j in range(WIN):
            pltpu.make_async_copy(zeros_hbm.at[pl.ds(0, 128)],
                                  msg0.at[par, j], sems.at[par]).wait()
            pltpu.make_async_copy(zeros_hbm.at[pl.ds(0, 128)],
                                  msg1.at[par, j], sems.at[par]).wait()

    def win(w, slot, par, drain, load):
        if drain:
            drain_scat(par)
        if load:
            idx_load(w + 2, (slot + 2) % 4)
        idx_wait(slot)
        cps = []
        for j in range(WIN):
            cps.append(pltpu.async_copy(tab0.at[swin.at[slot, j]],
                                        msg0.at[par, j], semg))
            cps.append(pltpu.async_copy(tab1.at[swin.at[slot, j]],
                                        msg1.at[par, j], semg))
        for c in cps:
            c.wait()
        for j in range(WIN):
            pltpu.async_copy(msg0.at[par, j], acc0.at[dwin.at[slot, j]],
                             sems.at[par], add=True)
            pltpu.async_copy(msg1.at[par, j], acc1.at[dwin.at[slot, j]],
                             sems.at[par], add=True)

    idx_load(0, 0)
    idx_load(1, 1)
    win(0, 0, 0, drain=False, load=True)
    win(1, 1, 1, drain=False, load=True)

    def body(g, carry):
        w0 = 2 + 4 * g
        for b in range(4):
            win(w0 + b, (2 + b) % 4, b % 2, drain=True, load=True)
        return carry

    lax.fori_loop(0, (NWIN - 4) // 4, body, 0)
    win(NWIN - 2, (NWIN - 2) % 4, 0, drain=True, load=False)
    win(NWIN - 1, (NWIN - 1) % 4, 1, drain=True, load=False)
    drain_scat(0)
    drain_scat(1)

    plsc.subcore_barrier()
    base = cid * 2 * NPAD + sid * SLICE
    pltpu.sync_copy(acc0.at[sl], out_hbm.at[pl.ds(base, SLICE)])
    pltpu.sync_copy(acc1.at[sl], out_hbm.at[pl.ds(base + NPAD, SLICE)])


@jax.jit
def _agg_call(src_p, dst_p, tabc0, tabc1, zeros1):
    k = functools.partial(
        pl.kernel,
        out_type=jax.ShapeDtypeStruct((4 * NPAD,), jnp.float32),
        mesh=_mesh(),
        scratch_types=[
            pltpu.VMEM((4, WIN, 128), jnp.int32),
            pltpu.VMEM((4, WIN, 128), jnp.int32),
            pltpu.VMEM((2, WIN, 128), jnp.float32),
            pltpu.VMEM((2, WIN, 128), jnp.float32),
            pltpu.VMEM_SHARED((NPAD,), jnp.float32),
            pltpu.VMEM_SHARED((NPAD,), jnp.float32),
            pltpu.VMEM_SHARED((NPAD,), jnp.float32),
            pltpu.VMEM_SHARED((NPAD,), jnp.float32),
            pltpu.SemaphoreType.DMA((4,)),
            pltpu.SemaphoreType.DMA,
            pltpu.SemaphoreType.DMA((2,)),
        ],
    )(_agg_body)
    return k(src_p, dst_p, tabc0, tabc1, zeros1).reshape(2, 2, NPAD)


def _dense1_body(degp_ref, xt_ref, dis_ref, s1_ref):
    deg = degp_ref[0:1, :] + degp_ref[1:2, :] + 1.0
    dis = lax.rsqrt(deg)
    dis_ref[...] = dis
    s1_ref[...] = xt_ref[...] * dis


@jax.jit
def _dense1(degp, xt):
    return pl.pallas_call(
        _dense1_body,
        out_shape=(jax.ShapeDtypeStruct((1, NPAD), jnp.float32),
                   jax.ShapeDtypeStruct((2, NPAD), jnp.float32)),
    )(degp, xt)


def _dense2_body(ap_ref, s1_ref, dis_ref, w1_ref, b1_ref, w2_ref, out_ref):
    dis = dis_ref[...]
    a = (ap_ref[0] + ap_ref[1] + s1_ref[...]) * dis
    a0 = a[0:1, :]
    a1 = a[1:2, :]
    t0 = jnp.zeros_like(a0)
    t1 = jnp.zeros_like(a0)
    for j in range(16):
        hj = jnp.maximum(a0 * w1_ref[0, j] + a1 * w1_ref[1, j] + b1_ref[j],
                         0.0)
        t0 = t0 + hj * w2_ref[j, 0]
        t1 = t1 + hj * w2_ref[j, 1]
    out_ref[0:1, :] = t0 * dis
    out_ref[1:2, :] = t1 * dis


@jax.jit
def _dense2(ap, s1t, dis, W1, b1, W2):
    return pl.pallas_call(
        _dense2_body,
        in_specs=[
            pl.BlockSpec(memory_space=pltpu.VMEM),
            pl.BlockSpec(memory_space=pltpu.VMEM),
            pl.BlockSpec(memory_space=pltpu.VMEM),
            pl.BlockSpec(memory_space=pltpu.SMEM),
            pl.BlockSpec(memory_space=pltpu.SMEM),
            pl.BlockSpec(memory_space=pltpu.SMEM),
        ],
        out_shape=jax.ShapeDtypeStruct((2, NPAD), jnp.float32),
    )(ap, s1t, dis, W1, b1, W2)


def _dense3_body(ap_ref, s2_ref, dis_ref, b2_ref, out_ref):
    z = (ap_ref[0] + ap_ref[1] + s2_ref[...]) * dis_ref[...]
    z0 = z[0:1, :] + b2_ref[0]
    z1 = z[1:2, :] + b2_ref[1]
    m = jnp.maximum(z0, z1)
    e0 = jnp.exp(z0 - m)
    e1 = jnp.exp(z1 - m)
    s = e0 + e1
    out_ref[0:1, :] = e0 / s
    out_ref[1:2, :] = e1 / s


@jax.jit
def _dense3(ap, s2t, dis, b2):
    return pl.pallas_call(
        _dense3_body,
        in_specs=[
            pl.BlockSpec(memory_space=pltpu.VMEM),
            pl.BlockSpec(memory_space=pltpu.VMEM),
            pl.BlockSpec(memory_space=pltpu.VMEM),
            pl.BlockSpec(memory_space=pltpu.SMEM),
        ],
        out_shape=jax.ShapeDtypeStruct((2, NPAD), jnp.float32),
    )(ap, s2t, dis, b2)


def kernel(x, edge_index, W1, b1, W2, b2):
    src = edge_index[0].astype(jnp.int32)
    dst = edge_index[1].astype(jnp.int32)
    pad_ids = (N_NODES
               + jnp.arange(EPAD - N_EDGES, dtype=jnp.int32)
               % (NPAD - N_NODES))
    src_p = jnp.concatenate([src, pad_ids]).reshape(EROWS, 128)
    dst_p = jnp.concatenate([dst, pad_ids]).reshape(EROWS, 128)
    xt = jnp.pad(x, ((0, NPAD - N_NODES), (0, 0))).T
    zeros1 = jnp.zeros((NPAD,), jnp.float32)
    ones128 = jnp.ones((128,), jnp.float32)

    degp = _deg_call(dst_p, ones128, zeros1)
    dis, s1t = _dense1(degp, xt)
    agg1 = _agg_call(src_p, dst_p, s1t[0], s1t[1], zeros1)
    s2t = _dense2(agg1, s1t, dis, W1, b1, W2)
    agg2 = _agg_call(src_p, dst_p, s2t[0], s2t[1], zeros1)
    outt = _dense3(agg2, s2t, dis, b2)
    return outt.T[:N_NODES]
